# baseline (speedup 1.0000x reference)
import jax
import jax.numpy as jnp
from jax import lax
from jax.experimental import pallas as pl
from jax.experimental.pallas import tpu as pltpu

_D_MODEL = 4096
_NUM_EXPERTS = 64
_BLOCK_T = 1024


def _router_kernel(x_ref, wr_ref, wg_ref, s_ref, g_ref):
    x = x_ref[...]
    logits = lax.dot_general(
        x, wr_ref[...], (((1,), (1,)), ((), ())),
        preferred_element_type=jnp.float32)
    m = jnp.max(logits, axis=-1, keepdims=True)
    e = jnp.exp(logits - m)
    s_ref[...] = e / jnp.sum(e, axis=-1, keepdims=True)
    g_logit = jnp.sum(x * wg_ref[...], axis=-1, keepdims=True)
    g_ref[...] = jax.nn.sigmoid(g_logit)


def kernel(x, W_router, W_shared_gate):
    tokens, d = x.shape
    n_exp = W_router.shape[0]

    grid = (tokens // _BLOCK_T,)
    s, g = pl.pallas_call(
        _router_kernel,
        grid=grid,
        in_specs=[
            pl.BlockSpec((_BLOCK_T, d), lambda i: (i, 0)),
            pl.BlockSpec((n_exp, d), lambda i: (0, 0)),
            pl.BlockSpec((1, d), lambda i: (0, 0)),
        ],
        out_specs=[
            pl.BlockSpec((_BLOCK_T, n_exp), lambda i: (i, 0)),
            pl.BlockSpec((_BLOCK_T, 1), lambda i: (i, 0)),
        ],
        out_shape=[
            jax.ShapeDtypeStruct((tokens, n_exp), x.dtype),
            jax.ShapeDtypeStruct((tokens, 1), x.dtype),
        ],
        compiler_params=pltpu.CompilerParams(
            dimension_semantics=("arbitrary",),
        ),
    )(x, W_router, W_shared_gate)
    return (s, g)
